# Initial kernel scaffold; baseline (speedup 1.0000x reference)
import jax
import jax.numpy as jnp
from jax.experimental import pallas as pl


def kernel(pred, target):
    raise NotImplementedError("write your pallas kernel here")


# speedup vs baseline: 23.2037x; 23.2037x over previous
import functools

import jax
import jax.numpy as jnp
import numpy as np
from jax.experimental import pallas as pl
from jax.experimental.pallas import tpu as pltpu

_IGNORE = 255
_THRESH = 0.7
_MIN_KEPT = 100000
_C = 19
_B, _H, _W = 8, 512, 512
_CH = 128
_STEPS = _B * (_H // _CH)
_ROWS = _STEPS * _CH


def _ohem_kernel(pred_ref, tgt_ref, tfloor_ref, out_ref, nll_ref):
    b = pl.program_id(0)
    h = pl.program_id(1)
    step = b * (_H // _CH) + h

    tgt = tgt_ref[0]
    x0 = pred_ref[0, 0]
    m = x0
    for c in range(1, _C):
        m = jnp.maximum(m, pred_ref[0, c])
    z = jnp.exp(x0 - m)
    tl = jnp.where(tgt == 0, x0, 0.0)
    for c in range(1, _C):
        xc = pred_ref[0, c]
        z = z + jnp.exp(xc - m)
        tl = tl + jnp.where(tgt == c, xc, 0.0)
    nll = m + jnp.log(z) - tl
    key = jnp.where(tgt != _IGNORE, nll, -1.0)
    nll_ref[pl.ds(step * _CH, _CH), :] = key

    @pl.when(jnp.logical_and(b == _B - 1, h == _H // _CH - 1))
    def _select():
        keys = nll_ref[:, :]
        valid = keys >= 0.0
        num_valid = jnp.sum(valid.astype(jnp.int32))
        bits = jax.lax.bitcast_convert_type(keys, jnp.uint32)
        bits = jnp.where(valid, bits, jnp.uint32(0))

        kplus1 = jnp.int32(_MIN_KEPT + 1)

        def body(i, v):
            shift = jnp.uint32(31) - i.astype(jnp.uint32)
            cand = v | jax.lax.shift_left(jnp.uint32(1), shift)
            cnt = jnp.sum((bits >= cand).astype(jnp.int32))
            return jnp.where(cnt >= kplus1, cand, v)

        vstar = jax.lax.fori_loop(0, 32, body, jnp.uint32(0))
        kth = jax.lax.bitcast_convert_type(vstar, jnp.float32)
        thr = jnp.maximum(kth, tfloor_ref[0])

        keep = jnp.logical_and(valid, keys >= thr)
        kept_cnt = jnp.maximum(jnp.sum(keep.astype(jnp.float32)), 1.0)
        kept_sum = jnp.sum(jnp.where(keep, keys, 0.0))
        all_cnt = jnp.maximum(num_valid.astype(jnp.float32), 1.0)
        all_sum = jnp.sum(jnp.where(valid, keys, 0.0))
        out_ref[0] = jnp.where(num_valid > jnp.int32(_MIN_KEPT),
                               kept_sum / kept_cnt, all_sum / all_cnt)


@jax.jit
def kernel(pred, target):
    tfloor = -jnp.log(jnp.float32(_THRESH)).reshape(1)
    out = pl.pallas_call(
        _ohem_kernel,
        grid=(_B, _H // _CH),
        in_specs=[
            pl.BlockSpec((1, _C, _CH, _W), lambda b, h: (b, 0, h, 0)),
            pl.BlockSpec((1, _CH, _W), lambda b, h: (b, h, 0)),
            pl.BlockSpec(memory_space=pltpu.SMEM),
        ],
        out_specs=pl.BlockSpec(memory_space=pltpu.SMEM),
        out_shape=jax.ShapeDtypeStruct((1,), jnp.float32),
        scratch_shapes=[pltpu.VMEM((_ROWS, _W), jnp.float32)],
    )(pred, target, tfloor)
    return out[0]
